# baseline (speedup 1.0000x reference)
import jax
import jax.numpy as jnp
from jax import lax
from jax.experimental import pallas as pl
from jax.experimental.pallas import tpu as pltpu
from jax.experimental.pallas import tpu_sc as plsc

_XR, _XC = 262144, 256
_ZR, _ZC = 16384, 1024
_BX = 8192
_NW = 32
_ZROWS = _ZR // _NW
_ZCH = 32
_NCH = _ZROWS // _ZCH
_NB = 3


def _x_kernel(x_ref, y_ref, xo_ref):
    i = pl.program_id(0)
    xo_ref[...] = x_ref[...]

    @pl.when(i == 0)
    def _patch():
        xo_ref[pl.ds(10, 1), :] = y_ref[pl.ds(0, 1), :]
        xo_ref[pl.ds(2, 1), :] = y_ref[pl.ds(1, 1), :]
        xo_ref[pl.ds(1, 1), :] = jnp.full((1, _XC), 45.0, jnp.float32)


def _z_body(z_hbm, w_hbm, zo_hbm, bufs, wv, isem, osem, wsem):
    wid = lax.axis_index("s") * 2 + lax.axis_index("c")
    base = wid * _ZROWS
    ins, outs = [], []

    def start_in(k):
        b = k % _NB
        c = pltpu.make_async_copy(z_hbm.at[pl.ds(base + k * _ZCH, _ZCH), :],
                                  bufs.at[b], isem.at[b])
        c.start()
        ins.append(c)

    for k in range(_NB):
        start_in(k)

    @pl.when(wid == 0)
    def _stage_w():
        cw = pltpu.make_async_copy(w_hbm, wv.at[pl.ds(0, 3)], wsem)
        cw.start()
        cw.wait()

    for k in range(_NCH):
        b = k % _NB
        ins[k].wait()
        if k == 0:
            @pl.when(wid == 0)
            def _patch():
                lane = lax.broadcasted_iota(jnp.int32, (16,), 0)
                wvv = wv[...]
                w0s, w1s, w2s = wvv[0], wvv[1], wvv[2]
                zero = jnp.zeros((16,), jnp.float32)
                row0 = bufs[0, 0, pl.ds(0, 16)]
                bufs[0, 0, pl.ds(0, 16)] = (row0
                                            + jnp.where(lane == 1, w2s, zero)
                                            + jnp.where(lane == 2, w1s, zero))
                row1 = bufs[0, 1, pl.ds(0, 16)]
                bufs[0, 1, pl.ds(0, 16)] = row1 + jnp.where(lane == 3, w0s, zero)
        co = pltpu.make_async_copy(bufs.at[b],
                                   zo_hbm.at[pl.ds(base + k * _ZCH, _ZCH), :],
                                   osem.at[b])
        co.start()
        outs.append(co)
        if k + _NB < _NCH:
            outs[k].wait()
            start_in(k + _NB)
    for k in range(_NCH - _NB, _NCH):
        outs[k].wait()


def kernel(x, y, z, w):
    zc = pl.kernel(
        _z_body,
        mesh=plsc.VectorSubcoreMesh(core_axis_name="c", subcore_axis_name="s"),
        out_type=jax.ShapeDtypeStruct((_ZR, _ZC), jnp.float32),
        scratch_types=[
            pltpu.VMEM((_NB, _ZCH, _ZC), jnp.float32),
            pltpu.VMEM((16,), jnp.float32),
            pltpu.SemaphoreType.DMA((_NB,)),
            pltpu.SemaphoreType.DMA((_NB,)),
            pltpu.SemaphoreType.DMA,
        ],
    )
    zo = zc(z, w)
    xo = pl.pallas_call(
        _x_kernel,
        grid=(_XR // _BX,),
        in_specs=[
            pl.BlockSpec((_BX, _XC), lambda i: (i, 0)),
            pl.BlockSpec((2, _XC), lambda i: (0, 0)),
        ],
        out_specs=pl.BlockSpec((_BX, _XC), lambda i: (i, 0)),
        out_shape=jax.ShapeDtypeStruct((_XR, _XC), jnp.float32),
    )(x, y)
    return (xo, zo)


# speedup vs baseline: 1.0856x; 1.0856x over previous
import jax
import jax.numpy as jnp
from jax.experimental import pallas as pl
from jax.experimental.pallas import tpu as pltpu

_XR, _XC = 262144, 256
_ZR, _ZC = 16384, 1024
_BX = 8192
_BZ = 512
_GRID = _XR // _BX


def _copy_patch_kernel(x_ref, y_ref, z_ref, w_ref, xo_ref, zo_ref):
    i = pl.program_id(0)
    xo_ref[...] = x_ref[...]
    zo_ref[...] = z_ref[...]

    @pl.when(i == 0)
    def _patch():
        xo_ref[pl.ds(10, 1), :] = y_ref[pl.ds(0, 1), :]
        xo_ref[pl.ds(2, 1), :] = y_ref[pl.ds(1, 1), :]
        xo_ref[pl.ds(1, 1), :] = jnp.full((1, _XC), 45.0, jnp.float32)
        zrows = z_ref[pl.ds(0, 2), :]
        row = jax.lax.broadcasted_iota(jnp.int32, (2, _ZC), 0)
        col = jax.lax.broadcasted_iota(jnp.int32, (2, _ZC), 1)
        add = (jnp.where((row == 1) & (col == 3), w_ref[0], 0.0)
               + jnp.where((row == 0) & (col == 2), w_ref[1], 0.0)
               + jnp.where((row == 0) & (col == 1), w_ref[2], 0.0))
        zo_ref[pl.ds(0, 2), :] = zrows + add


def kernel(x, y, z, w):
    xo, zo = pl.pallas_call(
        _copy_patch_kernel,
        grid=(_GRID,),
        in_specs=[
            pl.BlockSpec((_BX, _XC), lambda i: (i, 0)),
            pl.BlockSpec((2, _XC), lambda i: (0, 0)),
            pl.BlockSpec((_BZ, _ZC), lambda i: (i, 0)),
            pl.BlockSpec(memory_space=pltpu.SMEM),
        ],
        out_specs=[
            pl.BlockSpec((_BX, _XC), lambda i: (i, 0)),
            pl.BlockSpec((_BZ, _ZC), lambda i: (i, 0)),
        ],
        out_shape=[
            jax.ShapeDtypeStruct((_XR, _XC), jnp.float32),
            jax.ShapeDtypeStruct((_ZR, _ZC), jnp.float32),
        ],
    )(x, y, z, w)
    return (xo, zo)
